# baseline (speedup 1.0000x reference)
import numpy as np

import jax
import jax.numpy as jnp
from jax.experimental import pallas as pl
from jax.experimental.pallas import tpu as pltpu

_N = 22
_BB = 8
_NL = _N * (_N - 1) // 2

_U = np.zeros((_N, _NL), np.float32)
_V = np.zeros((_NL, _N), np.float32)
for _r in range(1, _N):
    for _q in range(_r):
        _j = _r * (_r - 1) // 2 + _q
        _U[_r, _j] = 1.0
        _V[_j, _q] = 1.0
_UT = np.ascontiguousarray(_U.T)
_VT = np.ascontiguousarray(_V.T)


def _make_kernel(n_batch, n_channels, seq_len):
    def _fused_kernel(ewp_ref, u_ref, v_ref, ut_ref, vt_ref, x_ref,
                      y_ref, ew_ref):
        ewp = ewp_ref[...]
        ewp_b = jnp.broadcast_to(ewp, (_N, _NL))
        hi = jax.lax.Precision.HIGHEST
        low = jnp.dot(u_ref[...] * ewp_b, v_ref[...],
                      preferred_element_type=jnp.float32, precision=hi)
        low_t = jnp.dot(vt_ref[...] * ewp_b, ut_ref[...],
                        preferred_element_type=jnp.float32, precision=hi)
        eye = (jax.lax.broadcasted_iota(jnp.int32, (_N, _N), 0) ==
               jax.lax.broadcasted_iota(jnp.int32, (_N, _N), 1)
               ).astype(jnp.float32)
        a = low + low_t + eye
        deg_row = jnp.sum(a, axis=1, keepdims=True)
        deg_col = jnp.sum(a, axis=0, keepdims=True)
        ew = jax.lax.rsqrt(deg_row) * a * jax.lax.rsqrt(deg_col) + eye
        ew_ref[...] = ew
        w2 = jnp.dot(ew, ew, preferred_element_type=jnp.float32)
        for i in range(n_batch):
            for c in range(n_channels):
                y_ref[i, :, c * seq_len:(c + 1) * seq_len] = jnp.dot(
                    w2, x_ref[i, :, c, :], preferred_element_type=jnp.float32)
    return _fused_kernel


def kernel(x, edge_weight_param):
    B, C, N, T = x.shape
    assert N == _N

    ewp2d = edge_weight_param.astype(jnp.float32)[None, :]
    xt = jnp.transpose(x, (0, 2, 1, 3))
    bb = _BB if B % _BB == 0 else 1

    y, ew = pl.pallas_call(
        _make_kernel(bb, C, T),
        out_shape=(
            jax.ShapeDtypeStruct((B, N, C * T), x.dtype),
            jax.ShapeDtypeStruct((N, N), jnp.float32),
        ),
        grid=(B // bb,),
        in_specs=[
            pl.BlockSpec((1, _NL), lambda b: (0, 0)),
            pl.BlockSpec((_N, _NL), lambda b: (0, 0)),
            pl.BlockSpec((_NL, _N), lambda b: (0, 0)),
            pl.BlockSpec((_NL, _N), lambda b: (0, 0)),
            pl.BlockSpec((_N, _NL), lambda b: (0, 0)),
            pl.BlockSpec((bb, N, C, T), lambda b: (b, 0, 0, 0)),
        ],
        out_specs=(
            pl.BlockSpec((bb, N, C * T), lambda b: (b, 0, 0)),
            pl.BlockSpec((N, N), lambda b: (0, 0)),
        ),
        compiler_params=pltpu.CompilerParams(
            dimension_semantics=("parallel",),
        ),
    )(ewp2d, jnp.asarray(_U), jnp.asarray(_V), jnp.asarray(_UT),
      jnp.asarray(_VT), xt)

    return y, ew


# speedup vs baseline: 1.7011x; 1.7011x over previous
import numpy as np

import jax
import jax.numpy as jnp
from jax.experimental import pallas as pl
from jax.experimental.pallas import tpu as pltpu

_N = 22
_BB = 16
_NL = _N * (_N - 1) // 2

_U = np.zeros((_N, _NL), np.float32)
_V = np.zeros((_NL, _N), np.float32)
for _r in range(1, _N):
    for _q in range(_r):
        _j = _r * (_r - 1) // 2 + _q
        _U[_r, _j] = 1.0
        _V[_j, _q] = 1.0
_UT = np.ascontiguousarray(_U.T)
_VT = np.ascontiguousarray(_V.T)

_G = 8 * _N
_Q = np.zeros((_G, _N), np.float32)
_R = np.zeros((_N, _G), np.float32)
_MSK = np.zeros((_G, _G), np.float32)
for _n in range(_N):
    for _b in range(8):
        _Q[_n * 8 + _b, _n] = 1.0
for _j in range(8):
    for _m in range(_N):
        _R[_m, _j * _N + _m] = 1.0
for _n in range(_N):
    for _b in range(8):
        for _m in range(_N):
            _MSK[_n * 8 + _b, _b * _N + _m] = 1.0


def _make_kernel(n_batch, n_channels, seq_len):
    def _fused_kernel(ewp_ref, u_ref, v_ref, ut_ref, vt_ref, q_ref, r_ref,
                      msk_ref, x_ref, y_ref, ew_ref):
        ewp = ewp_ref[...]
        ewp_b = jnp.broadcast_to(ewp, (_N, _NL))
        hi = jax.lax.Precision.HIGHEST
        low = jnp.dot(u_ref[...] * ewp_b, v_ref[...],
                      preferred_element_type=jnp.float32, precision=hi)
        low_t = jnp.dot(vt_ref[...] * ewp_b, ut_ref[...],
                        preferred_element_type=jnp.float32, precision=hi)
        eye = (jax.lax.broadcasted_iota(jnp.int32, (_N, _N), 0) ==
               jax.lax.broadcasted_iota(jnp.int32, (_N, _N), 1)
               ).astype(jnp.float32)
        a = low + low_t + eye
        deg_row = jnp.sum(a, axis=1, keepdims=True)
        deg_col = jnp.sum(a, axis=0, keepdims=True)
        ew = jax.lax.rsqrt(deg_row) * a * jax.lax.rsqrt(deg_col) + eye
        ew_ref[...] = ew
        w2 = jnp.dot(ew, ew, preferred_element_type=jnp.float32)
        w2r = jnp.dot(w2, r_ref[...], preferred_element_type=jnp.float32)
        swcat = jnp.dot(q_ref[...], w2r,
                        preferred_element_type=jnp.float32) * msk_ref[...]
        for base in range(0, n_batch, 8):
            for c in range(n_channels):
                xs = x_ref[base:base + 8, :, c, :].reshape(_G, seq_len)
                z = jnp.dot(swcat, xs,
                            preferred_element_type=jnp.float32)
                y_ref[:, base:base + 8,
                      c * seq_len:(c + 1) * seq_len] = z.reshape(
                          _N, 8, seq_len)
    return _fused_kernel


def kernel(x, edge_weight_param):
    B, C, N, T = x.shape
    assert N == _N

    ewp2d = edge_weight_param.astype(jnp.float32)[None, :]
    xt = jnp.transpose(x, (0, 2, 1, 3))
    bb = _BB if B % _BB == 0 else 8
    assert B % bb == 0

    yt, ew = pl.pallas_call(
        _make_kernel(bb, C, T),
        out_shape=(
            jax.ShapeDtypeStruct((N, B, C * T), x.dtype),
            jax.ShapeDtypeStruct((N, N), jnp.float32),
        ),
        grid=(B // bb,),
        in_specs=[
            pl.BlockSpec((1, _NL), lambda b: (0, 0)),
            pl.BlockSpec((_N, _NL), lambda b: (0, 0)),
            pl.BlockSpec((_NL, _N), lambda b: (0, 0)),
            pl.BlockSpec((_NL, _N), lambda b: (0, 0)),
            pl.BlockSpec((_N, _NL), lambda b: (0, 0)),
            pl.BlockSpec((_G, _N), lambda b: (0, 0)),
            pl.BlockSpec((_N, _G), lambda b: (0, 0)),
            pl.BlockSpec((_G, _G), lambda b: (0, 0)),
            pl.BlockSpec((bb, N, C, T), lambda b: (b, 0, 0, 0)),
        ],
        out_specs=(
            pl.BlockSpec((N, bb, C * T), lambda b: (0, b, 0)),
            pl.BlockSpec((N, N), lambda b: (0, 0)),
        ),
        compiler_params=pltpu.CompilerParams(
            dimension_semantics=("parallel",),
        ),
    )(ewp2d, jnp.asarray(_U), jnp.asarray(_V), jnp.asarray(_UT),
      jnp.asarray(_VT), jnp.asarray(_Q), jnp.asarray(_R), jnp.asarray(_MSK),
      xt)

    return jnp.transpose(yt, (1, 0, 2)), ew
